# baseline (speedup 1.0000x reference)
import functools

import jax
import jax.numpy as jnp
from jax import lax
from jax.experimental import pallas as pl
from jax.experimental.pallas import tpu as pltpu
from jax.experimental.pallas import tpu_sc as plsc

_B = 8
_NP = 1024
_C = 128
_K = 64
_R = 0.15
_H1 = 128
_H2 = 128
_BNP = _B * _NP

_TI = 128
_NI = _NP // _TI
_BS_ITERS = 50
_NEG = -1e30

_NC = 2
_NS = 16
_NW = _NC * _NS
_ROWS_PER_W = _BNP // _NW
_RB = 8
_NB = _ROWS_PER_W // _RB


def _prologue_kernel(x_ref, pos_ref, frac_ref, fracT_ref, tv_ref, r2_ref,
                     w1x_ref, w1p_ref, b1_ref,
                     a_ref, c_ref, sl_ref, cnt_ref):
    x = x_ref[0]
    pos = pos_ref[0]
    f = frac_ref[0]
    ft = fracT_ref[0]
    tv = tv_ref[0]
    r2 = r2_ref[0][0, 0]
    w1p = w1p_ref[...]
    b1 = b1_ref[...]

    pc = (pos[:, 0:1] * w1p[0:1, :]
          + pos[:, 1:2] * w1p[1:2, :]
          + pos[:, 2:3] * w1p[2:3, :])
    a = jnp.dot(x.astype(jnp.bfloat16), w1x_ref[...].astype(jnp.bfloat16),
                preferred_element_type=jnp.float32) + pc + b1
    a_ref[0] = a
    c_ref[0] = -pc

    bf = jnp.bfloat16
    f32 = jnp.float32
    d0 = (f[:, 0:1] - ft[0:1, :]).astype(bf).astype(f32)
    d1 = (f[:, 1:2] - ft[1:2, :]).astype(bf).astype(f32)
    d2c = (f[:, 2:3] - ft[2:3, :]).astype(bf).astype(f32)
    tvb = tv.astype(bf).astype(f32)
    dsq = None
    for l in range(3):
        dv = d0 * tvb[0, l] + d1 * tvb[1, l] + d2c * tvb[2, l]
        dsq = dv * dv if dsq is None else dsq + dv * dv

    rowmax = jnp.max(dsq, axis=1, keepdims=True)
    lo = jnp.zeros_like(rowmax)
    hi = rowmax

    def body(_, carry):
        lo, hi = carry
        mid = 0.5 * (lo + hi)
        cnt = jnp.sum((dsq <= mid).astype(f32), axis=1, keepdims=True)
        ge = cnt >= float(_K)
        return jnp.where(ge, lo, mid), jnp.where(ge, mid, hi)

    lo, hi = lax.fori_loop(0, _BS_ITERS, body, (lo, hi))

    mask = jnp.logical_and(dsq <= hi, dsq < r2)
    m = mask.astype(f32)
    csum = m
    sh = 1
    while sh < _NP:
        csum = csum + jnp.concatenate(
            [jnp.zeros((_NP, sh), f32), csum[:, :_NP - sh]], axis=1)
        sh *= 2
    sl = jnp.where(mask, csum - 1.0, -1.0)
    sl = jnp.where(sl > float(_K - 1), -1.0, sl)
    sl_ref[0] = sl.astype(jnp.int32)
    cnt_ref[0] = jnp.minimum(csum[:, _NP - 1:], float(_K)).astype(jnp.int32)


def _sc_gather_kernel(sl_hbm, a_hbm, out_hbm, slbuf, idxbuf, rows_v, sem):
    wid = lax.axis_index("s") * _NC + lax.axis_index("c")

    def prefill(q, _):
        idxbuf[pl.ds(q * 16, 16)] = jnp.zeros((16,), jnp.int32)
        return 0

    lax.fori_loop(0, _RB * _K // 16, prefill, 0)

    def batch_body(it, _):
        row0 = wid * _ROWS_PER_W + it * _RB
        pltpu.sync_copy(sl_hbm.at[pl.ds(row0 * _NP, _RB * _NP)], slbuf)
        jbase = (row0 // _NP) * _NP

        def chunk_body(t, _):
            rr = t // (_NP // 16)
            cchunk = t % (_NP // 16)
            v = slbuf[pl.ds(t * 16, 16)]
            jv = lax.iota(jnp.int32, 16) + (cchunk * 16 + jbase)
            plsc.store_scatter(idxbuf, [rr * _K + v], jv, mask=v >= 0)
            return 0

        lax.fori_loop(0, _RB * (_NP // 16), chunk_body, 0)
        pltpu.async_copy(a_hbm.at[idxbuf], rows_v, sem).wait()
        pltpu.sync_copy(rows_v, out_hbm.at[pl.ds(row0 * _K, _RB * _K)])
        return 0

    lax.fori_loop(0, _NB, batch_body, 0)


def _pair_kernel(ag_ref, c_ref, cnt_ref, w2_ref, b2_ref, out_ref):
    ag = ag_ref[0]
    ci = c_ref[0]
    cnt3 = cnt_ref[0]

    h1 = jnp.maximum(ag + ci[:, None, :], 0.0)
    h1f = h1.reshape(_TI * _K, _H1).astype(jnp.bfloat16)
    h2 = jnp.dot(h1f, w2_ref[...].astype(jnp.bfloat16),
                 preferred_element_type=jnp.float32)
    h2 = jnp.maximum(h2 + b2_ref[...], 0.0).reshape(_TI, _K, _H2)

    kio = lax.broadcasted_iota(jnp.int32, (_TI, _K, 1), 1)
    h2 = jnp.where(kio < cnt3, h2, _NEG)
    out_ref[0] = jnp.max(h2, axis=1)


def kernel(x, pos, fps_pos, batch, frac_pos, trans_vec, scale, W1, b1, W2,
           b2):
    del fps_pos, batch
    xg = x.reshape(_B, _NP, _C)
    posg = pos.reshape(_B, _NP, 3)
    fracg = frac_pos.reshape(_B, _NP, 3)
    fracT = fracg.transpose(0, 2, 1)
    w1x = W1[:_C]
    w1p = W1[_C:]
    b1r = b1.reshape(1, _H1)
    b2r = b2.reshape(1, _H2)
    r2 = ((_R / scale) ** 2).reshape(_B, 1, 1).astype(jnp.float32)

    a, c, sl, cnt = pl.pallas_call(
        _prologue_kernel,
        grid=(_B,),
        in_specs=[
            pl.BlockSpec((1, _NP, _C), lambda b: (b, 0, 0)),
            pl.BlockSpec((1, _NP, 3), lambda b: (b, 0, 0)),
            pl.BlockSpec((1, _NP, 3), lambda b: (b, 0, 0)),
            pl.BlockSpec((1, 3, _NP), lambda b: (b, 0, 0)),
            pl.BlockSpec((1, 3, 3), lambda b: (b, 0, 0)),
            pl.BlockSpec((1, 1, 1), lambda b: (b, 0, 0)),
            pl.BlockSpec((_C, _H1), lambda b: (0, 0)),
            pl.BlockSpec((3, _H1), lambda b: (0, 0)),
            pl.BlockSpec((1, _H1), lambda b: (0, 0)),
        ],
        out_specs=[
            pl.BlockSpec((1, _NP, _H1), lambda b: (b, 0, 0)),
            pl.BlockSpec((1, _NP, _H1), lambda b: (b, 0, 0)),
            pl.BlockSpec((1, _NP, _NP), lambda b: (b, 0, 0)),
            pl.BlockSpec((1, _NP, 1), lambda b: (b, 0, 0)),
        ],
        out_shape=[
            jax.ShapeDtypeStruct((_B, _NP, _H1), jnp.float32),
            jax.ShapeDtypeStruct((_B, _NP, _H1), jnp.float32),
            jax.ShapeDtypeStruct((_B, _NP, _NP), jnp.int32),
            jax.ShapeDtypeStruct((_B, _NP, 1), jnp.int32),
        ],
    )(xg, posg, fracg, fracT, trans_vec, r2, w1x, w1p, b1r)

    sc_gather = functools.partial(
        pl.kernel,
        out_type=jax.ShapeDtypeStruct((_BNP * _K, _H1), jnp.float32),
        mesh=plsc.VectorSubcoreMesh(core_axis_name="c",
                                    subcore_axis_name="s"),
        scratch_types=[
            pltpu.VMEM((_RB * _NP,), jnp.int32),
            pltpu.VMEM((_RB * _K,), jnp.int32),
            pltpu.VMEM((_RB * _K, _H1), jnp.float32),
            pltpu.SemaphoreType.DMA,
        ],
        compiler_params=pltpu.CompilerParams(needs_layout_passes=False),
    )(_sc_gather_kernel)

    ag = sc_gather(sl.reshape(_BNP * _NP), a.reshape(_BNP, _H1))
    ag = ag.reshape(_B, _NP, _K, _H1)

    out = pl.pallas_call(
        _pair_kernel,
        grid=(_B, _NI),
        in_specs=[
            pl.BlockSpec((1, _TI, _K, _H1), lambda b, i: (b, i, 0, 0)),
            pl.BlockSpec((1, _TI, _H1), lambda b, i: (b, i, 0)),
            pl.BlockSpec((1, _TI, 1, 1), lambda b, i: (b, i, 0, 0)),
            pl.BlockSpec((_H1, _H2), lambda b, i: (0, 0)),
            pl.BlockSpec((1, _H2), lambda b, i: (0, 0)),
        ],
        out_specs=pl.BlockSpec((1, _TI, _H2), lambda b, i: (b, i, 0)),
        out_shape=jax.ShapeDtypeStruct((_B, _NP, _H2), jnp.float32),
        compiler_params=pltpu.CompilerParams(
            dimension_semantics=("parallel", "parallel")),
    )(ag, c, cnt.reshape(_B, _NP, 1, 1), W2, b2r)

    return out.reshape(_BNP, _H2)


# speedup vs baseline: 1.0003x; 1.0003x over previous
import functools

import jax
import jax.numpy as jnp
from jax import lax
from jax.experimental import pallas as pl
from jax.experimental.pallas import tpu as pltpu
from jax.experimental.pallas import tpu_sc as plsc

_B = 8
_NP = 1024
_C = 128
_K = 64
_R = 0.15
_H1 = 128
_H2 = 128
_BNP = _B * _NP

_TI = 128
_NI = _NP // _TI
_BS_ITERS = 50
_NEG = -1e30

_NC = 2
_NS = 16
_NW = _NC * _NS
_ROWS_PER_W = _BNP // _NW
_RB = 4
_NB = _ROWS_PER_W // _RB


def _prologue_kernel(x_ref, pos_ref, frac_ref, fracT_ref, tv_ref, r2_ref,
                     w1x_ref, w1p_ref, b1_ref,
                     a_ref, c_ref, sl_ref, cnt_ref):
    x = x_ref[0]
    pos = pos_ref[0]
    f = frac_ref[0]
    ft = fracT_ref[0]
    tv = tv_ref[0]
    r2 = r2_ref[0][0, 0]
    w1p = w1p_ref[...]
    b1 = b1_ref[...]

    pc = (pos[:, 0:1] * w1p[0:1, :]
          + pos[:, 1:2] * w1p[1:2, :]
          + pos[:, 2:3] * w1p[2:3, :])
    a = jnp.dot(x.astype(jnp.bfloat16), w1x_ref[...].astype(jnp.bfloat16),
                preferred_element_type=jnp.float32) + pc + b1
    a_ref[0] = a
    c_ref[0] = -pc

    bf = jnp.bfloat16
    f32 = jnp.float32
    d0 = (f[:, 0:1] - ft[0:1, :]).astype(bf).astype(f32)
    d1 = (f[:, 1:2] - ft[1:2, :]).astype(bf).astype(f32)
    d2c = (f[:, 2:3] - ft[2:3, :]).astype(bf).astype(f32)
    tvb = tv.astype(bf).astype(f32)
    dsq = None
    for l in range(3):
        dv = d0 * tvb[0, l] + d1 * tvb[1, l] + d2c * tvb[2, l]
        dsq = dv * dv if dsq is None else dsq + dv * dv

    rowmax = jnp.max(dsq, axis=1, keepdims=True)
    lo = jnp.zeros_like(rowmax)
    hi = rowmax

    def body(_, carry):
        lo, hi = carry
        mid = 0.5 * (lo + hi)
        cnt = jnp.sum((dsq <= mid).astype(f32), axis=1, keepdims=True)
        ge = cnt >= float(_K)
        return jnp.where(ge, lo, mid), jnp.where(ge, mid, hi)

    lo, hi = lax.fori_loop(0, _BS_ITERS, body, (lo, hi))

    mask = jnp.logical_and(dsq <= hi, dsq < r2)
    m = mask.astype(f32)
    csum = m
    sh = 1
    while sh < _NP:
        csum = csum + jnp.concatenate(
            [jnp.zeros((_NP, sh), f32), csum[:, :_NP - sh]], axis=1)
        sh *= 2
    sl = jnp.where(mask, csum - 1.0, -1.0)
    sl = jnp.where(sl > float(_K - 1), -1.0, sl)
    sl_ref[0] = sl.astype(jnp.int32)
    cnt_ref[0] = jnp.minimum(csum[:, _NP - 1:], float(_K)).astype(jnp.int32)


def _sc_gather_kernel(sl_hbm, a_hbm, out_hbm,
                      slb0, slb1, idx0, idx1, rv0, rv1,
                      ssem0, ssem1, gsem, osem0, osem1):
    wid = lax.axis_index("s") * _NC + lax.axis_index("c")
    row_base = wid * _ROWS_PER_W
    slbufs = (slb0, slb1)
    idxs = (idx0, idx1)
    rvs = (rv0, rv1)
    ssems = (ssem0, ssem1)
    osems = (osem0, osem1)

    for q in range(2):
        for p in range(_RB * _K // 16):
            idxs[q][pl.ds(p * 16, 16)] = jnp.zeros((16,), jnp.int32)

    pltpu.async_copy(sl_hbm.at[pl.ds(row_base * _NP, _RB * _NP)], slb0,
                     ssem0)

    def gloop(g, _):
        for q in range(2):
            it = g * 2 + q
            row0 = row_base + it * _RB
            pltpu.make_async_copy(
                sl_hbm.at[pl.ds(0, _RB * _NP)], slbufs[q],
                ssems[q]).wait()

            @pl.when(it + 1 < _NB)
            def _prefetch():
                nrow0 = row_base + (it + 1) * _RB
                pltpu.async_copy(
                    sl_hbm.at[pl.ds(nrow0 * _NP, _RB * _NP)],
                    slbufs[1 - q], ssems[1 - q])

            jbase = (row0 // _NP) * _NP

            @plsc.parallel_loop(0, _RB * (_NP // 16), unroll=8)
            def _build(t):
                rr = t // (_NP // 16)
                cc = t % (_NP // 16)
                off = pl.multiple_of(t * 16, 16)
                v = slbufs[q][pl.ds(off, 16)]
                jv = lax.iota(jnp.int32, 16) + (cc * 16 + jbase)
                plsc.store_scatter(idxs[q], [rr * _K + v], jv,
                                   mask=v >= 0)

            @pl.when(it >= 2)
            def _drain():
                pltpu.make_async_copy(
                    rvs[q], out_hbm.at[pl.ds(0, _RB * _K)],
                    osems[q]).wait()

            pltpu.async_copy(a_hbm.at[idxs[q]], rvs[q], gsem).wait()
            pltpu.async_copy(rvs[q], out_hbm.at[pl.ds(row0 * _K, _RB * _K)],
                             osems[q])
        return 0

    lax.fori_loop(0, _NB // 2, gloop, 0)
    for q in range(2):
        pltpu.make_async_copy(rvs[q], out_hbm.at[pl.ds(0, _RB * _K)],
                              osems[q]).wait()


def _pair_kernel(ag_ref, c_ref, cnt_ref, w2_ref, b2_ref, out_ref):
    ag = ag_ref[0]
    ci = c_ref[0]
    cnt3 = cnt_ref[0]

    h1 = jnp.maximum(ag + ci[:, None, :], 0.0)
    h1f = h1.reshape(_TI * _K, _H1).astype(jnp.bfloat16)
    h2 = jnp.dot(h1f, w2_ref[...].astype(jnp.bfloat16),
                 preferred_element_type=jnp.float32)
    h2 = jnp.maximum(h2 + b2_ref[...], 0.0).reshape(_TI, _K, _H2)

    kio = lax.broadcasted_iota(jnp.int32, (_TI, _K, 1), 1)
    h2 = jnp.where(kio < cnt3, h2, _NEG)
    out_ref[0] = jnp.max(h2, axis=1)


def kernel(x, pos, fps_pos, batch, frac_pos, trans_vec, scale, W1, b1, W2,
           b2):
    del fps_pos, batch
    xg = x.reshape(_B, _NP, _C)
    posg = pos.reshape(_B, _NP, 3)
    fracg = frac_pos.reshape(_B, _NP, 3)
    fracT = fracg.transpose(0, 2, 1)
    w1x = W1[:_C]
    w1p = W1[_C:]
    b1r = b1.reshape(1, _H1)
    b2r = b2.reshape(1, _H2)
    r2 = ((_R / scale) ** 2).reshape(_B, 1, 1).astype(jnp.float32)

    a, c, sl, cnt = pl.pallas_call(
        _prologue_kernel,
        grid=(_B,),
        in_specs=[
            pl.BlockSpec((1, _NP, _C), lambda b: (b, 0, 0)),
            pl.BlockSpec((1, _NP, 3), lambda b: (b, 0, 0)),
            pl.BlockSpec((1, _NP, 3), lambda b: (b, 0, 0)),
            pl.BlockSpec((1, 3, _NP), lambda b: (b, 0, 0)),
            pl.BlockSpec((1, 3, 3), lambda b: (b, 0, 0)),
            pl.BlockSpec((1, 1, 1), lambda b: (b, 0, 0)),
            pl.BlockSpec((_C, _H1), lambda b: (0, 0)),
            pl.BlockSpec((3, _H1), lambda b: (0, 0)),
            pl.BlockSpec((1, _H1), lambda b: (0, 0)),
        ],
        out_specs=[
            pl.BlockSpec((1, _NP, _H1), lambda b: (b, 0, 0)),
            pl.BlockSpec((1, _NP, _H1), lambda b: (b, 0, 0)),
            pl.BlockSpec((1, _NP, _NP), lambda b: (b, 0, 0)),
            pl.BlockSpec((1, _NP, 1), lambda b: (b, 0, 0)),
        ],
        out_shape=[
            jax.ShapeDtypeStruct((_B, _NP, _H1), jnp.float32),
            jax.ShapeDtypeStruct((_B, _NP, _H1), jnp.float32),
            jax.ShapeDtypeStruct((_B, _NP, _NP), jnp.int32),
            jax.ShapeDtypeStruct((_B, _NP, 1), jnp.int32),
        ],
    )(xg, posg, fracg, fracT, trans_vec, r2, w1x, w1p, b1r)

    sc_gather = functools.partial(
        pl.kernel,
        out_type=jax.ShapeDtypeStruct((_BNP * _K, _H1), jnp.float32),
        mesh=plsc.VectorSubcoreMesh(core_axis_name="c",
                                    subcore_axis_name="s"),
        scratch_types=[
            pltpu.VMEM((_RB * _NP,), jnp.int32),
            pltpu.VMEM((_RB * _NP,), jnp.int32),
            pltpu.VMEM((_RB * _K,), jnp.int32),
            pltpu.VMEM((_RB * _K,), jnp.int32),
            pltpu.VMEM((_RB * _K, _H1), jnp.float32),
            pltpu.VMEM((_RB * _K, _H1), jnp.float32),
            pltpu.SemaphoreType.DMA,
            pltpu.SemaphoreType.DMA,
            pltpu.SemaphoreType.DMA,
            pltpu.SemaphoreType.DMA,
            pltpu.SemaphoreType.DMA,
        ],
        compiler_params=pltpu.CompilerParams(needs_layout_passes=False),
    )(_sc_gather_kernel)

    ag = sc_gather(sl.reshape(_BNP * _NP), a.reshape(_BNP, _H1))
    ag = ag.reshape(_B, _NP, _K, _H1)

    out = pl.pallas_call(
        _pair_kernel,
        grid=(_B, _NI),
        in_specs=[
            pl.BlockSpec((1, _TI, _K, _H1), lambda b, i: (b, i, 0, 0)),
            pl.BlockSpec((1, _TI, _H1), lambda b, i: (b, i, 0)),
            pl.BlockSpec((1, _TI, 1, 1), lambda b, i: (b, i, 0, 0)),
            pl.BlockSpec((_H1, _H2), lambda b, i: (0, 0)),
            pl.BlockSpec((1, _H2), lambda b, i: (0, 0)),
        ],
        out_specs=pl.BlockSpec((1, _TI, _H2), lambda b, i: (b, i, 0)),
        out_shape=jax.ShapeDtypeStruct((_B, _NP, _H2), jnp.float32),
        compiler_params=pltpu.CompilerParams(
            dimension_semantics=("parallel", "parallel")),
    )(ag, c, cnt.reshape(_B, _NP, 1, 1), W2, b2r)

    return out.reshape(_BNP, _H2)


# speedup vs baseline: 19.4385x; 19.4336x over previous
import functools

import jax
import jax.numpy as jnp
from jax import lax
from jax.experimental import pallas as pl
from jax.experimental.pallas import tpu as pltpu
from jax.experimental.pallas import tpu_sc as plsc

_B = 8
_NP = 1024
_C = 128
_K = 64
_R = 0.15
_H1 = 128
_H2 = 128
_BNP = _B * _NP

_TI = 128
_NI = _NP // _TI
_BS_ITERS = 50
_NEG = -1e30

_NC = 2
_NS = 16
_NW = _NC * _NS
_ROWS_PER_W = _BNP // _NW
_RB = 4
_NB = _ROWS_PER_W // _RB


def _prologue_kernel(x_ref, pos_ref, frac_ref, fracT_ref, tv_ref, r2_ref,
                     w1x_ref, w1p_ref, b1_ref,
                     a_ref, c_ref, sl_ref, cnt_ref):
    x = x_ref[0]
    pos = pos_ref[0]
    f = frac_ref[0]
    ft = fracT_ref[0]
    tv = tv_ref[0]
    r2 = r2_ref[0][0, 0]
    w1p = w1p_ref[...]
    b1 = b1_ref[...]

    pc = (pos[:, 0:1] * w1p[0:1, :]
          + pos[:, 1:2] * w1p[1:2, :]
          + pos[:, 2:3] * w1p[2:3, :])
    a = jnp.dot(x.astype(jnp.bfloat16), w1x_ref[...].astype(jnp.bfloat16),
                preferred_element_type=jnp.float32) + pc + b1
    a_ref[0] = a
    c_ref[0] = -pc

    bf = jnp.bfloat16
    f32 = jnp.float32
    d0 = (f[:, 0:1] - ft[0:1, :]).astype(bf).astype(f32)
    d1 = (f[:, 1:2] - ft[1:2, :]).astype(bf).astype(f32)
    d2c = (f[:, 2:3] - ft[2:3, :]).astype(bf).astype(f32)
    tvb = tv.astype(bf).astype(f32)
    dsq = None
    for l in range(3):
        dv = d0 * tvb[0, l] + d1 * tvb[1, l] + d2c * tvb[2, l]
        dsq = dv * dv if dsq is None else dsq + dv * dv

    rowmax = jnp.max(dsq, axis=1, keepdims=True)
    lo = jnp.zeros_like(rowmax)
    hi = rowmax

    def body(_, carry):
        lo, hi = carry
        mid = 0.5 * (lo + hi)
        cnt = jnp.sum((dsq <= mid).astype(f32), axis=1, keepdims=True)
        ge = cnt >= float(_K)
        return jnp.where(ge, lo, mid), jnp.where(ge, mid, hi)

    lo, hi = lax.fori_loop(0, _BS_ITERS, body, (lo, hi))

    mask = jnp.logical_and(dsq <= hi, dsq < r2)
    m = mask.astype(f32)
    csum = m
    sh = 1
    while sh < _NP:
        csum = csum + jnp.concatenate(
            [jnp.zeros((_NP, sh), f32), csum[:, :_NP - sh]], axis=1)
        sh *= 2
    sl = jnp.where(mask, csum - 1.0, -1.0)
    sl = jnp.where(sl > float(_K - 1), -1.0, sl)
    sl_ref[0] = sl.astype(jnp.int32)
    cnt_ref[0] = jnp.minimum(csum[:, _NP - 1:], float(_K)).astype(jnp.int32)


def _sc_gather_kernel(sl_hbm, a_hbm, out_hbm,
                      ashr, slb0, slb1, idx0, idx1, rv0, rv1,
                      ssem0, ssem1, gsem, osem0, osem1):
    cid = lax.axis_index("c")
    sid = lax.axis_index("s")
    half = _BNP // _NC

    @pl.when(sid == 0)
    def _stage():
        pltpu.sync_copy(a_hbm.at[pl.ds(cid * half, half)], ashr)

    plsc.subcore_barrier()

    wid = cid * _NS + sid
    row_base = wid * _ROWS_PER_W
    loc_off = cid * half
    slbufs = (slb0, slb1)
    idxs = (idx0, idx1)
    rvs = (rv0, rv1)
    ssems = (ssem0, ssem1)
    osems = (osem0, osem1)

    for q in range(2):
        for p in range(_RB * _K // 16):
            idxs[q][pl.ds(p * 16, 16)] = jnp.zeros((16,), jnp.int32)

    pltpu.async_copy(sl_hbm.at[pl.ds(row_base * _NP, _RB * _NP)], slb0,
                     ssem0)

    def gloop(g, _):
        for q in range(2):
            it = g * 2 + q
            row0 = row_base + it * _RB
            pltpu.make_async_copy(
                sl_hbm.at[pl.ds(0, _RB * _NP)], slbufs[q],
                ssems[q]).wait()

            @pl.when(it + 1 < _NB)
            def _prefetch():
                nrow0 = row_base + (it + 1) * _RB
                pltpu.async_copy(
                    sl_hbm.at[pl.ds(nrow0 * _NP, _RB * _NP)],
                    slbufs[1 - q], ssems[1 - q])

            jbase = (row0 // _NP) * _NP - loc_off

            @plsc.parallel_loop(0, _RB * (_NP // 16), unroll=8)
            def _build(t):
                rr = t // (_NP // 16)
                cc = t % (_NP // 16)
                off = pl.multiple_of(t * 16, 16)
                v = slbufs[q][pl.ds(off, 16)]
                jv = lax.iota(jnp.int32, 16) + (cc * 16 + jbase)
                plsc.store_scatter(idxs[q], [rr * _K + v], jv,
                                   mask=v >= 0)

            @pl.when(it >= 2)
            def _drain():
                pltpu.make_async_copy(
                    rvs[q], out_hbm.at[pl.ds(0, _RB * _K)],
                    osems[q]).wait()

            pltpu.async_copy(ashr.at[idxs[q]], rvs[q], gsem).wait()
            pltpu.async_copy(rvs[q], out_hbm.at[pl.ds(row0 * _K, _RB * _K)],
                             osems[q])
        return 0

    lax.fori_loop(0, _NB // 2, gloop, 0)
    for q in range(2):
        pltpu.make_async_copy(rvs[q], out_hbm.at[pl.ds(0, _RB * _K)],
                              osems[q]).wait()


def _pair_kernel(ag_ref, c_ref, cnt_ref, w2_ref, b2_ref, out_ref):
    ag = ag_ref[0]
    ci = c_ref[0]
    cnt3 = cnt_ref[0]

    h1 = jnp.maximum(ag + ci[:, None, :], 0.0)
    h1f = h1.reshape(_TI * _K, _H1).astype(jnp.bfloat16)
    h2 = jnp.dot(h1f, w2_ref[...].astype(jnp.bfloat16),
                 preferred_element_type=jnp.float32)
    h2 = jnp.maximum(h2 + b2_ref[...], 0.0).reshape(_TI, _K, _H2)

    kio = lax.broadcasted_iota(jnp.int32, (_TI, _K, 1), 1)
    h2 = jnp.where(kio < cnt3, h2, _NEG)
    out_ref[0] = jnp.max(h2, axis=1)


def kernel(x, pos, fps_pos, batch, frac_pos, trans_vec, scale, W1, b1, W2,
           b2):
    del fps_pos, batch
    xg = x.reshape(_B, _NP, _C)
    posg = pos.reshape(_B, _NP, 3)
    fracg = frac_pos.reshape(_B, _NP, 3)
    fracT = fracg.transpose(0, 2, 1)
    w1x = W1[:_C]
    w1p = W1[_C:]
    b1r = b1.reshape(1, _H1)
    b2r = b2.reshape(1, _H2)
    r2 = ((_R / scale) ** 2).reshape(_B, 1, 1).astype(jnp.float32)

    a, c, sl, cnt = pl.pallas_call(
        _prologue_kernel,
        grid=(_B,),
        in_specs=[
            pl.BlockSpec((1, _NP, _C), lambda b: (b, 0, 0)),
            pl.BlockSpec((1, _NP, 3), lambda b: (b, 0, 0)),
            pl.BlockSpec((1, _NP, 3), lambda b: (b, 0, 0)),
            pl.BlockSpec((1, 3, _NP), lambda b: (b, 0, 0)),
            pl.BlockSpec((1, 3, 3), lambda b: (b, 0, 0)),
            pl.BlockSpec((1, 1, 1), lambda b: (b, 0, 0)),
            pl.BlockSpec((_C, _H1), lambda b: (0, 0)),
            pl.BlockSpec((3, _H1), lambda b: (0, 0)),
            pl.BlockSpec((1, _H1), lambda b: (0, 0)),
        ],
        out_specs=[
            pl.BlockSpec((1, _NP, _H1), lambda b: (b, 0, 0)),
            pl.BlockSpec((1, _NP, _H1), lambda b: (b, 0, 0)),
            pl.BlockSpec((1, _NP, _NP), lambda b: (b, 0, 0)),
            pl.BlockSpec((1, _NP, 1), lambda b: (b, 0, 0)),
        ],
        out_shape=[
            jax.ShapeDtypeStruct((_B, _NP, _H1), jnp.float32),
            jax.ShapeDtypeStruct((_B, _NP, _H1), jnp.float32),
            jax.ShapeDtypeStruct((_B, _NP, _NP), jnp.int32),
            jax.ShapeDtypeStruct((_B, _NP, 1), jnp.int32),
        ],
    )(xg, posg, fracg, fracT, trans_vec, r2, w1x, w1p, b1r)

    sc_gather = functools.partial(
        pl.kernel,
        out_type=jax.ShapeDtypeStruct((_BNP * _K, _H1), jnp.float32),
        mesh=plsc.VectorSubcoreMesh(core_axis_name="c",
                                    subcore_axis_name="s"),
        scratch_types=[
            pltpu.VMEM_SHARED((_BNP // _NC, _H1), jnp.float32),
            pltpu.VMEM((_RB * _NP,), jnp.int32),
            pltpu.VMEM((_RB * _NP,), jnp.int32),
            pltpu.VMEM((_RB * _K,), jnp.int32),
            pltpu.VMEM((_RB * _K,), jnp.int32),
            pltpu.VMEM((_RB * _K, _H1), jnp.float32),
            pltpu.VMEM((_RB * _K, _H1), jnp.float32),
            pltpu.SemaphoreType.DMA,
            pltpu.SemaphoreType.DMA,
            pltpu.SemaphoreType.DMA,
            pltpu.SemaphoreType.DMA,
            pltpu.SemaphoreType.DMA,
        ],
        compiler_params=pltpu.CompilerParams(needs_layout_passes=False),
    )(_sc_gather_kernel)

    ag = sc_gather(sl.reshape(_BNP * _NP), a.reshape(_BNP, _H1))
    ag = ag.reshape(_B, _NP, _K, _H1)

    out = pl.pallas_call(
        _pair_kernel,
        grid=(_B, _NI),
        in_specs=[
            pl.BlockSpec((1, _TI, _K, _H1), lambda b, i: (b, i, 0, 0)),
            pl.BlockSpec((1, _TI, _H1), lambda b, i: (b, i, 0)),
            pl.BlockSpec((1, _TI, 1, 1), lambda b, i: (b, i, 0, 0)),
            pl.BlockSpec((_H1, _H2), lambda b, i: (0, 0)),
            pl.BlockSpec((1, _H2), lambda b, i: (0, 0)),
        ],
        out_specs=pl.BlockSpec((1, _TI, _H2), lambda b, i: (b, i, 0)),
        out_shape=jax.ShapeDtypeStruct((_B, _NP, _H2), jnp.float32),
        compiler_params=pltpu.CompilerParams(
            dimension_semantics=("parallel", "parallel")),
    )(ag, c, cnt.reshape(_B, _NP, 1, 1), W2, b2r)

    return out.reshape(_BNP, _H2)


# speedup vs baseline: 21.2344x; 1.0924x over previous
import functools

import jax
import jax.numpy as jnp
from jax import lax
from jax.experimental import pallas as pl
from jax.experimental.pallas import tpu as pltpu
from jax.experimental.pallas import tpu_sc as plsc

_B = 8
_NP = 1024
_C = 128
_K = 64
_R = 0.15
_H1 = 128
_H2 = 128
_BNP = _B * _NP

_TI = 128
_NI = _NP // _TI
_BS_ITERS = 50
_NEG = -1e30

_NC = 2
_NS = 16
_NW = _NC * _NS
_ROWS_CALL = _NP // _NW
_RB = 4
_NB_CALL = _ROWS_CALL // _RB


def _prologue_kernel(x_ref, pos_ref, frac_ref, fracT_ref, tv_ref, r2_ref,
                     w1x_ref, w1p_ref, b1_ref,
                     a_ref, c_ref, sl_ref, cnt_ref):
    x = x_ref[0]
    pos = pos_ref[0]
    f = frac_ref[0]
    ft = fracT_ref[0]
    tv = tv_ref[0]
    r2 = r2_ref[0][0, 0]
    w1p = w1p_ref[...]
    b1 = b1_ref[...]

    pc = (pos[:, 0:1] * w1p[0:1, :]
          + pos[:, 1:2] * w1p[1:2, :]
          + pos[:, 2:3] * w1p[2:3, :])
    a = jnp.dot(x.astype(jnp.bfloat16), w1x_ref[...].astype(jnp.bfloat16),
                preferred_element_type=jnp.float32) + pc + b1
    a_ref[0] = a
    c_ref[0] = -pc

    bf = jnp.bfloat16
    f32 = jnp.float32
    d0 = (f[:, 0:1] - ft[0:1, :]).astype(bf).astype(f32)
    d1 = (f[:, 1:2] - ft[1:2, :]).astype(bf).astype(f32)
    d2c = (f[:, 2:3] - ft[2:3, :]).astype(bf).astype(f32)
    tvb = tv.astype(bf).astype(f32)
    dsq = None
    for l in range(3):
        dv = d0 * tvb[0, l] + d1 * tvb[1, l] + d2c * tvb[2, l]
        dsq = dv * dv if dsq is None else dsq + dv * dv

    rowmax = jnp.max(dsq, axis=1, keepdims=True)
    lo = jnp.zeros_like(rowmax)
    hi = rowmax

    def body(_, carry):
        lo, hi = carry
        mid = 0.5 * (lo + hi)
        cnt = jnp.sum((dsq <= mid).astype(f32), axis=1, keepdims=True)
        ge = cnt >= float(_K)
        return jnp.where(ge, lo, mid), jnp.where(ge, mid, hi)

    lo, hi = lax.fori_loop(0, _BS_ITERS, body, (lo, hi))

    mask = jnp.logical_and(dsq <= hi, dsq < r2)
    m = mask.astype(f32)
    csum = m
    sh = 1
    while sh < _NP:
        csum = csum + jnp.concatenate(
            [jnp.zeros((_NP, sh), f32), csum[:, :_NP - sh]], axis=1)
        sh *= 2
    sl = jnp.where(mask, csum - 1.0, -1.0)
    sl = jnp.where(sl > float(_K - 1), -1.0, sl)
    sl_ref[0] = sl.astype(jnp.int32)
    cnt_ref[0] = jnp.minimum(csum[:, _NP - 1:], float(_K)).astype(jnp.int32)


def _sc_gather_kernel(sl_hbm, a_hbm, out_hbm,
                      ashr, slb0, slb1, idx0, idx1, rv0, rv1,
                      ssem0, ssem1, gsem, osem0, osem1):
    cid = lax.axis_index("c")
    sid = lax.axis_index("s")

    @pl.when(sid == 0)
    def _stage():
        pltpu.sync_copy(a_hbm, ashr)

    plsc.subcore_barrier()

    wid = cid * _NS + sid
    row_base = wid * _ROWS_CALL
    slbufs = (slb0, slb1)
    idxs = (idx0, idx1)
    rvs = (rv0, rv1)
    ssems = (ssem0, ssem1)
    osems = (osem0, osem1)

    for q in range(2):
        for p in range(_RB * _K // 16):
            idxs[q][pl.ds(p * 16, 16)] = jnp.zeros((16,), jnp.int32)

    pltpu.async_copy(sl_hbm.at[pl.ds(row_base * _NP, _RB * _NP)], slb0,
                     ssem0)

    def gloop(g, _):
        for q in range(2):
            it = g * 2 + q
            row0 = row_base + it * _RB
            pltpu.make_async_copy(
                sl_hbm.at[pl.ds(0, _RB * _NP)], slbufs[q],
                ssems[q]).wait()

            @pl.when(it + 1 < _NB_CALL)
            def _prefetch():
                nrow0 = row_base + (it + 1) * _RB
                pltpu.async_copy(
                    sl_hbm.at[pl.ds(nrow0 * _NP, _RB * _NP)],
                    slbufs[1 - q], ssems[1 - q])

            jbase = 0

            @plsc.parallel_loop(0, _RB * (_NP // 16), unroll=8)
            def _build(t):
                rr = t // (_NP // 16)
                cc = t % (_NP // 16)
                off = pl.multiple_of(t * 16, 16)
                v = slbufs[q][pl.ds(off, 16)]
                jv = lax.iota(jnp.int32, 16) + (cc * 16 + jbase)
                plsc.store_scatter(idxs[q], [rr * _K + v], jv,
                                   mask=v >= 0)

            @pl.when(it >= 2)
            def _drain():
                pltpu.make_async_copy(
                    rvs[q], out_hbm.at[pl.ds(0, _RB * _K)],
                    osems[q]).wait()

            pltpu.async_copy(ashr.at[idxs[q]], rvs[q], gsem).wait()
            pltpu.async_copy(rvs[q], out_hbm.at[pl.ds(row0 * _K, _RB * _K)],
                             osems[q])
        return 0

    lax.fori_loop(0, _NB_CALL // 2, gloop, 0)
    for q in range(2):
        pltpu.make_async_copy(rvs[q], out_hbm.at[pl.ds(0, _RB * _K)],
                              osems[q]).wait()


def _pair_kernel(ag_ref, c_ref, cnt_ref, w2_ref, b2_ref, out_ref):
    ag = ag_ref[0]
    ci = c_ref[0]
    cnt3 = cnt_ref[0]

    h1 = jnp.maximum(ag + ci[:, None, :], 0.0)
    h1f = h1.reshape(_TI * _K, _H1).astype(jnp.bfloat16)
    h2 = jnp.dot(h1f, w2_ref[...].astype(jnp.bfloat16),
                 preferred_element_type=jnp.float32)
    h2 = jnp.maximum(h2 + b2_ref[...], 0.0).reshape(_TI, _K, _H2)

    kio = lax.broadcasted_iota(jnp.int32, (_TI, _K, 1), 1)
    h2 = jnp.where(kio < cnt3, h2, _NEG)
    out_ref[0] = jnp.max(h2, axis=1)


def kernel(x, pos, fps_pos, batch, frac_pos, trans_vec, scale, W1, b1, W2,
           b2):
    del fps_pos, batch
    xg = x.reshape(_B, _NP, _C)
    posg = pos.reshape(_B, _NP, 3)
    fracg = frac_pos.reshape(_B, _NP, 3)
    fracT = fracg.transpose(0, 2, 1)
    w1x = W1[:_C]
    w1p = W1[_C:]
    b1r = b1.reshape(1, _H1)
    b2r = b2.reshape(1, _H2)
    r2 = ((_R / scale) ** 2).reshape(_B, 1, 1).astype(jnp.float32)

    a, c, sl, cnt = pl.pallas_call(
        _prologue_kernel,
        grid=(_B,),
        in_specs=[
            pl.BlockSpec((1, _NP, _C), lambda b: (b, 0, 0)),
            pl.BlockSpec((1, _NP, 3), lambda b: (b, 0, 0)),
            pl.BlockSpec((1, _NP, 3), lambda b: (b, 0, 0)),
            pl.BlockSpec((1, 3, _NP), lambda b: (b, 0, 0)),
            pl.BlockSpec((1, 3, 3), lambda b: (b, 0, 0)),
            pl.BlockSpec((1, 1, 1), lambda b: (b, 0, 0)),
            pl.BlockSpec((_C, _H1), lambda b: (0, 0)),
            pl.BlockSpec((3, _H1), lambda b: (0, 0)),
            pl.BlockSpec((1, _H1), lambda b: (0, 0)),
        ],
        out_specs=[
            pl.BlockSpec((1, _NP, _H1), lambda b: (b, 0, 0)),
            pl.BlockSpec((1, _NP, _H1), lambda b: (b, 0, 0)),
            pl.BlockSpec((1, _NP, _NP), lambda b: (b, 0, 0)),
            pl.BlockSpec((1, _NP, 1), lambda b: (b, 0, 0)),
        ],
        out_shape=[
            jax.ShapeDtypeStruct((_B, _NP, _H1), jnp.float32),
            jax.ShapeDtypeStruct((_B, _NP, _H1), jnp.float32),
            jax.ShapeDtypeStruct((_B, _NP, _NP), jnp.int32),
            jax.ShapeDtypeStruct((_B, _NP, 1), jnp.int32),
        ],
    )(xg, posg, fracg, fracT, trans_vec, r2, w1x, w1p, b1r)

    sc_gather = functools.partial(
        pl.kernel,
        out_type=jax.ShapeDtypeStruct((_NP * _K, _H1), jnp.float32),
        mesh=plsc.VectorSubcoreMesh(core_axis_name="c",
                                    subcore_axis_name="s"),
        scratch_types=[
            pltpu.VMEM_SHARED((_NP, _H1), jnp.float32),
            pltpu.VMEM((_RB * _NP,), jnp.int32),
            pltpu.VMEM((_RB * _NP,), jnp.int32),
            pltpu.VMEM((_RB * _K,), jnp.int32),
            pltpu.VMEM((_RB * _K,), jnp.int32),
            pltpu.VMEM((_RB * _K, _H1), jnp.float32),
            pltpu.VMEM((_RB * _K, _H1), jnp.float32),
            pltpu.SemaphoreType.DMA,
            pltpu.SemaphoreType.DMA,
            pltpu.SemaphoreType.DMA,
            pltpu.SemaphoreType.DMA,
            pltpu.SemaphoreType.DMA,
        ],
        compiler_params=pltpu.CompilerParams(needs_layout_passes=False),
    )(_sc_gather_kernel)

    pair = functools.partial(
        pl.pallas_call,
        _pair_kernel,
        grid=(_NI,),
        in_specs=[
            pl.BlockSpec((1, _TI, _K, _H1), lambda i: (0, i, 0, 0)),
            pl.BlockSpec((1, _TI, _H1), lambda i: (0, i, 0)),
            pl.BlockSpec((1, _TI, 1, 1), lambda i: (0, i, 0, 0)),
            pl.BlockSpec((_H1, _H2), lambda i: (0, 0)),
            pl.BlockSpec((1, _H2), lambda i: (0, 0)),
        ],
        out_specs=pl.BlockSpec((1, _TI, _H2), lambda i: (0, i, 0)),
        out_shape=jax.ShapeDtypeStruct((1, _NP, _H2), jnp.float32),
        compiler_params=pltpu.CompilerParams(
            dimension_semantics=("parallel",)),
    )()

    cnt4 = cnt.reshape(_B, _NP, 1, 1)
    outs = []
    for b in range(_B):
        ag_b = sc_gather(sl[b].reshape(_NP * _NP), a[b])
        ag_b = ag_b.reshape(1, _NP, _K, _H1)
        outs.append(pair(ag_b, c[b:b + 1], cnt4[b:b + 1], W2, b2r))

    return jnp.concatenate(outs, axis=0).reshape(_BNP, _H2)


# speedup vs baseline: 23.0342x; 1.0848x over previous
import functools

import jax
import jax.numpy as jnp
from jax import lax
from jax.experimental import pallas as pl
from jax.experimental.pallas import tpu as pltpu
from jax.experimental.pallas import tpu_sc as plsc

_B = 8
_NP = 1024
_C = 128
_K = 64
_R = 0.15
_H1 = 128
_H2 = 128
_BNP = _B * _NP

_TI = 128
_NI = _NP // _TI
_BS_ITERS = 36
_NEG = -1e30

_NC = 2
_NS = 16
_NW = _NC * _NS
_ROWS_CALL = _NP // _NW
_RB = 4
_NB_CALL = _ROWS_CALL // _RB


def _prologue_kernel(x_ref, pos_ref, frac_ref, fracT_ref, tv_ref, r2_ref,
                     w1x_ref, w1p_ref, b1_ref,
                     a_ref, c_ref, sl_ref, cnt_ref):
    x = x_ref[0]
    pos = pos_ref[0]
    f = frac_ref[0]
    ft = fracT_ref[0]
    tv = tv_ref[0]
    r2 = r2_ref[0][0, 0]
    w1p = w1p_ref[...]
    b1 = b1_ref[...]

    pc = (pos[:, 0:1] * w1p[0:1, :]
          + pos[:, 1:2] * w1p[1:2, :]
          + pos[:, 2:3] * w1p[2:3, :])
    a = jnp.dot(x.astype(jnp.bfloat16), w1x_ref[...].astype(jnp.bfloat16),
                preferred_element_type=jnp.float32) + pc + b1
    a_ref[0] = a
    c_ref[0] = -pc

    bf = jnp.bfloat16
    f32 = jnp.float32
    d0 = (f[:, 0:1] - ft[0:1, :]).astype(bf).astype(f32)
    d1 = (f[:, 1:2] - ft[1:2, :]).astype(bf).astype(f32)
    d2c = (f[:, 2:3] - ft[2:3, :]).astype(bf).astype(f32)
    tvb = tv.astype(bf).astype(f32)
    dsq = None
    for l in range(3):
        dv = d0 * tvb[0, l] + d1 * tvb[1, l] + d2c * tvb[2, l]
        dsq = dv * dv if dsq is None else dsq + dv * dv

    lo = jnp.zeros((_NP, 1), jnp.float32)
    hi = jnp.full((_NP, 1), r2, jnp.float32)

    def body(_, carry):
        lo, hi = carry
        mid = 0.5 * (lo + hi)
        cnt = jnp.sum((dsq <= mid).astype(f32), axis=1, keepdims=True)
        ge = cnt >= float(_K)
        return jnp.where(ge, lo, mid), jnp.where(ge, mid, hi)

    lo, hi = lax.fori_loop(0, _BS_ITERS, body, (lo, hi))

    mask = jnp.logical_and(dsq <= hi, dsq < r2)
    m = mask.astype(f32)
    csum = m
    sh = 1
    while sh < _NP:
        csum = csum + jnp.concatenate(
            [jnp.zeros((_NP, sh), f32), csum[:, :_NP - sh]], axis=1)
        sh *= 2
    sl = jnp.where(mask, csum - 1.0, -1.0)
    sl = jnp.where(sl > float(_K - 1), -1.0, sl)
    sl_ref[0] = sl.astype(jnp.int32)
    cnt_ref[0] = jnp.minimum(csum[:, _NP - 1:], float(_K)).astype(jnp.int32)


def _sc_gather_kernel(sl_hbm, a_hbm, out_hbm,
                      ashr, slb0, slb1, idx0, idx1, rv0, rv1,
                      ssem0, ssem1, gsem, osem0, osem1):
    cid = lax.axis_index("c")
    sid = lax.axis_index("s")

    @pl.when(sid == 0)
    def _stage():
        pltpu.sync_copy(a_hbm, ashr)

    plsc.subcore_barrier()

    wid = cid * _NS + sid
    row_base = wid * _ROWS_CALL
    slbufs = (slb0, slb1)
    idxs = (idx0, idx1)
    rvs = (rv0, rv1)
    ssems = (ssem0, ssem1)
    osems = (osem0, osem1)

    for q in range(2):
        for p in range(_RB * _K // 16):
            idxs[q][pl.ds(p * 16, 16)] = jnp.zeros((16,), jnp.int32)

    pltpu.async_copy(sl_hbm.at[pl.ds(row_base * _NP, _RB * _NP)], slb0,
                     ssem0)

    def gloop(g, _):
        for q in range(2):
            it = g * 2 + q
            row0 = row_base + it * _RB
            pltpu.make_async_copy(
                sl_hbm.at[pl.ds(0, _RB * _NP)], slbufs[q],
                ssems[q]).wait()

            @pl.when(it + 1 < _NB_CALL)
            def _prefetch():
                nrow0 = row_base + (it + 1) * _RB
                pltpu.async_copy(
                    sl_hbm.at[pl.ds(nrow0 * _NP, _RB * _NP)],
                    slbufs[1 - q], ssems[1 - q])

            jbase = 0

            @plsc.parallel_loop(0, _RB * (_NP // 16), unroll=8)
            def _build(t):
                rr = t // (_NP // 16)
                cc = t % (_NP // 16)
                off = pl.multiple_of(t * 16, 16)
                v = slbufs[q][pl.ds(off, 16)]
                jv = lax.iota(jnp.int32, 16) + (cc * 16 + jbase)
                plsc.store_scatter(idxs[q], [rr * _K + v], jv,
                                   mask=v >= 0)

            @pl.when(it >= 2)
            def _drain():
                pltpu.make_async_copy(
                    rvs[q], out_hbm.at[pl.ds(0, _RB * _K)],
                    osems[q]).wait()

            pltpu.async_copy(ashr.at[idxs[q]], rvs[q], gsem).wait()
            pltpu.async_copy(rvs[q], out_hbm.at[pl.ds(row0 * _K, _RB * _K)],
                             osems[q])
        return 0

    lax.fori_loop(0, _NB_CALL // 2, gloop, 0)
    for q in range(2):
        pltpu.make_async_copy(rvs[q], out_hbm.at[pl.ds(0, _RB * _K)],
                              osems[q]).wait()


def _pair_kernel(ag_ref, c_ref, cnt_ref, w2_ref, b2_ref, out_ref):
    ag = ag_ref[0]
    ci = c_ref[0]
    cnt3 = cnt_ref[0]

    h1 = jnp.maximum(ag + ci[:, None, :], 0.0)
    h1f = h1.reshape(_TI * _K, _H1).astype(jnp.bfloat16)
    h2 = jnp.dot(h1f, w2_ref[...].astype(jnp.bfloat16),
                 preferred_element_type=jnp.float32)
    h2 = jnp.maximum(h2 + b2_ref[...], 0.0).reshape(_TI, _K, _H2)

    kio = lax.broadcasted_iota(jnp.int32, (_TI, _K, 1), 1)
    h2 = jnp.where(kio < cnt3, h2, _NEG)
    out_ref[0] = jnp.max(h2, axis=1)


def kernel(x, pos, fps_pos, batch, frac_pos, trans_vec, scale, W1, b1, W2,
           b2):
    del fps_pos, batch
    xg = x.reshape(_B, _NP, _C)
    posg = pos.reshape(_B, _NP, 3)
    fracg = frac_pos.reshape(_B, _NP, 3)
    fracT = fracg.transpose(0, 2, 1)
    w1x = W1[:_C]
    w1p = W1[_C:]
    b1r = b1.reshape(1, _H1)
    b2r = b2.reshape(1, _H2)
    r2 = ((_R / scale) ** 2).reshape(_B, 1, 1).astype(jnp.float32)

    a, c, sl, cnt = pl.pallas_call(
        _prologue_kernel,
        grid=(_B,),
        in_specs=[
            pl.BlockSpec((1, _NP, _C), lambda b: (b, 0, 0)),
            pl.BlockSpec((1, _NP, 3), lambda b: (b, 0, 0)),
            pl.BlockSpec((1, _NP, 3), lambda b: (b, 0, 0)),
            pl.BlockSpec((1, 3, _NP), lambda b: (b, 0, 0)),
            pl.BlockSpec((1, 3, 3), lambda b: (b, 0, 0)),
            pl.BlockSpec((1, 1, 1), lambda b: (b, 0, 0)),
            pl.BlockSpec((_C, _H1), lambda b: (0, 0)),
            pl.BlockSpec((3, _H1), lambda b: (0, 0)),
            pl.BlockSpec((1, _H1), lambda b: (0, 0)),
        ],
        out_specs=[
            pl.BlockSpec((1, _NP, _H1), lambda b: (b, 0, 0)),
            pl.BlockSpec((1, _NP, _H1), lambda b: (b, 0, 0)),
            pl.BlockSpec((1, _NP, _NP), lambda b: (b, 0, 0)),
            pl.BlockSpec((1, _NP, 1), lambda b: (b, 0, 0)),
        ],
        out_shape=[
            jax.ShapeDtypeStruct((_B, _NP, _H1), jnp.float32),
            jax.ShapeDtypeStruct((_B, _NP, _H1), jnp.float32),
            jax.ShapeDtypeStruct((_B, _NP, _NP), jnp.int32),
            jax.ShapeDtypeStruct((_B, _NP, 1), jnp.int32),
        ],
    )(xg, posg, fracg, fracT, trans_vec, r2, w1x, w1p, b1r)

    sc_gather = functools.partial(
        pl.kernel,
        out_type=jax.ShapeDtypeStruct((_NP * _K, _H1), jnp.float32),
        mesh=plsc.VectorSubcoreMesh(core_axis_name="c",
                                    subcore_axis_name="s"),
        scratch_types=[
            pltpu.VMEM_SHARED((_NP, _H1), jnp.float32),
            pltpu.VMEM((_RB * _NP,), jnp.int32),
            pltpu.VMEM((_RB * _NP,), jnp.int32),
            pltpu.VMEM((_RB * _K,), jnp.int32),
            pltpu.VMEM((_RB * _K,), jnp.int32),
            pltpu.VMEM((_RB * _K, _H1), jnp.float32),
            pltpu.VMEM((_RB * _K, _H1), jnp.float32),
            pltpu.SemaphoreType.DMA,
            pltpu.SemaphoreType.DMA,
            pltpu.SemaphoreType.DMA,
            pltpu.SemaphoreType.DMA,
            pltpu.SemaphoreType.DMA,
        ],
        compiler_params=pltpu.CompilerParams(needs_layout_passes=False),
    )(_sc_gather_kernel)

    pair = functools.partial(
        pl.pallas_call,
        _pair_kernel,
        grid=(_NI,),
        in_specs=[
            pl.BlockSpec((1, _TI, _K, _H1), lambda i: (0, i, 0, 0)),
            pl.BlockSpec((1, _TI, _H1), lambda i: (0, i, 0)),
            pl.BlockSpec((1, _TI, 1, 1), lambda i: (0, i, 0, 0)),
            pl.BlockSpec((_H1, _H2), lambda i: (0, 0)),
            pl.BlockSpec((1, _H2), lambda i: (0, 0)),
        ],
        out_specs=pl.BlockSpec((1, _TI, _H2), lambda i: (0, i, 0)),
        out_shape=jax.ShapeDtypeStruct((1, _NP, _H2), jnp.float32),
        compiler_params=pltpu.CompilerParams(
            dimension_semantics=("parallel",)),
    )()

    cnt4 = cnt.reshape(_B, _NP, 1, 1)
    outs = []
    for b in range(_B):
        ag_b = sc_gather(sl[b].reshape(_NP * _NP), a[b])
        ag_b = ag_b.reshape(1, _NP, _K, _H1)
        outs.append(pair(ag_b, c[b:b + 1], cnt4[b:b + 1], W2, b2r))

    return jnp.concatenate(outs, axis=0).reshape(_BNP, _H2)
